# baseline (speedup 1.0000x reference)
import jax
import jax.numpy as jnp
from jax import lax
from jax.experimental import pallas as pl
from jax.experimental.pallas import tpu as pltpu
from jax.experimental.pallas import tpu_sc as plsc

D_MODEL = 2048
SEQ_LEN = 4096
BATCH = 4
WIDE = BATCH * D_MODEL
S_BLK = 128


def _gather_body(idx_hbm, pe_hbm, pos_hbm, idx_v, pos_v, gsem):
    cid = lax.axis_index("c")
    sid = lax.axis_index("s")

    @pl.when(jnp.logical_and(cid == 0, sid == 0))
    def _():
        pltpu.sync_copy(idx_hbm, idx_v)
        pltpu.async_copy(pe_hbm.at[idx_v], pos_v, gsem).wait()
        pltpu.sync_copy(pos_v, pos_hbm)


def _sc_gather(idx, pe):
    return pl.kernel(
        _gather_body,
        out_type=jax.ShapeDtypeStruct((BATCH, D_MODEL), jnp.float32),
        mesh=plsc.VectorSubcoreMesh(core_axis_name="c", subcore_axis_name="s"),
        scratch_types=[
            pltpu.VMEM((BATCH,), jnp.int32),
            pltpu.VMEM((BATCH, D_MODEL), jnp.float32),
            pltpu.SemaphoreType.DMA,
        ],
    )(idx, pe)


def _add_body(x_ref, pos_ref, o_ref):
    o_ref[...] = x_ref[...] + pos_ref[...]


def kernel(x, pe, idx):
    pos = _sc_gather(idx.astype(jnp.int32), pe)
    x2 = x.reshape(SEQ_LEN, WIDE)
    posf = pos.reshape(1, WIDE)
    out = pl.pallas_call(
        _add_body,
        grid=(SEQ_LEN // S_BLK,),
        in_specs=[
            pl.BlockSpec((S_BLK, WIDE), lambda i: (i, 0)),
            pl.BlockSpec((1, WIDE), lambda i: (0, 0)),
        ],
        out_specs=pl.BlockSpec((S_BLK, WIDE), lambda i: (i, 0)),
        out_shape=jax.ShapeDtypeStruct((SEQ_LEN, WIDE), jnp.float32),
    )(x2, posf)
    return out.reshape(SEQ_LEN, BATCH, D_MODEL)


# speedup vs baseline: 3.4493x; 3.4493x over previous
import jax
import jax.numpy as jnp
from jax import lax
from jax.experimental import pallas as pl
from jax.experimental.pallas import tpu as pltpu
from jax.experimental.pallas import tpu_sc as plsc

D_MODEL = 2048
SEQ_LEN = 4096
BATCH = 4
WIDE = BATCH * D_MODEL
S_BLK = 128


def _gather_body(idx_hbm, pe_hbm, pos_hbm, idx_v, pos_v, gsem):
    cid = lax.axis_index("c")
    sid = lax.axis_index("s")

    @pl.when(jnp.logical_and(cid == 0, sid == 0))
    def _():
        pltpu.sync_copy(idx_hbm, idx_v)
        pltpu.async_copy(pe_hbm.at[idx_v], pos_v, gsem).wait()
        pltpu.sync_copy(pos_v, pos_hbm)


def _sc_gather(idx, pe):
    return pl.kernel(
        _gather_body,
        out_type=jax.ShapeDtypeStruct((BATCH, D_MODEL), jnp.float32),
        mesh=plsc.VectorSubcoreMesh(core_axis_name="c", subcore_axis_name="s"),
        scratch_types=[
            pltpu.VMEM((BATCH,), jnp.int32),
            pltpu.VMEM((BATCH, D_MODEL), jnp.float32),
            pltpu.SemaphoreType.DMA,
        ],
    )(idx, pe)


def _add_body(x_ref, pos_ref, o_ref):
    o_ref[...] = x_ref[...] + pos_ref[...][None, :, :]


def kernel(x, pe, idx):
    pos = _sc_gather(idx.astype(jnp.int32), pe)
    return pl.pallas_call(
        _add_body,
        grid=(SEQ_LEN // S_BLK,),
        in_specs=[
            pl.BlockSpec((S_BLK, BATCH, D_MODEL), lambda i: (i, 0, 0)),
            pl.BlockSpec((BATCH, D_MODEL), lambda i: (0, 0)),
        ],
        out_specs=pl.BlockSpec((S_BLK, BATCH, D_MODEL), lambda i: (i, 0, 0)),
        out_shape=jax.ShapeDtypeStruct((SEQ_LEN, BATCH, D_MODEL), jnp.float32),
    )(x, pos)


# speedup vs baseline: 3.5101x; 1.0176x over previous
import jax
import jax.numpy as jnp
from jax import lax
from jax.experimental import pallas as pl
from jax.experimental.pallas import tpu as pltpu
from jax.experimental.pallas import tpu_sc as plsc

D_MODEL = 2048
SEQ_LEN = 4096
BATCH = 4
WIDE = BATCH * D_MODEL
S_BLK = 256


def _gather_body(idx_hbm, pe_hbm, pos_hbm, idx_v, pos_v, gsem):
    cid = lax.axis_index("c")
    sid = lax.axis_index("s")

    @pl.when(jnp.logical_and(cid == 0, sid == 0))
    def _():
        pltpu.sync_copy(idx_hbm, idx_v)
        pltpu.async_copy(pe_hbm.at[idx_v], pos_v, gsem).wait()
        pltpu.sync_copy(pos_v, pos_hbm)


def _sc_gather(idx, pe):
    return pl.kernel(
        _gather_body,
        out_type=jax.ShapeDtypeStruct((BATCH, D_MODEL), jnp.float32),
        mesh=plsc.VectorSubcoreMesh(core_axis_name="c", subcore_axis_name="s"),
        scratch_types=[
            pltpu.VMEM((BATCH,), jnp.int32),
            pltpu.VMEM((BATCH, D_MODEL), jnp.float32),
            pltpu.SemaphoreType.DMA,
        ],
    )(idx, pe)


def _add_body(x_ref, pos_ref, o_ref):
    o_ref[...] = x_ref[...] + pos_ref[...][None, :, :]


def kernel(x, pe, idx):
    pos = _sc_gather(idx.astype(jnp.int32), pe)
    return pl.pallas_call(
        _add_body,
        grid=(SEQ_LEN // S_BLK,),
        in_specs=[
            pl.BlockSpec((S_BLK, BATCH, D_MODEL), lambda i: (i, 0, 0)),
            pl.BlockSpec((BATCH, D_MODEL), lambda i: (0, 0)),
        ],
        out_specs=pl.BlockSpec((S_BLK, BATCH, D_MODEL), lambda i: (i, 0, 0)),
        out_shape=jax.ShapeDtypeStruct((SEQ_LEN, BATCH, D_MODEL), jnp.float32),
    )(x, pos)


# speedup vs baseline: 3.5449x; 1.0099x over previous
import jax
import jax.numpy as jnp
from jax import lax
from jax.experimental import pallas as pl
from jax.experimental.pallas import tpu as pltpu
from jax.experimental.pallas import tpu_sc as plsc

D_MODEL = 2048
SEQ_LEN = 4096
BATCH = 4
WIDE = BATCH * D_MODEL
S_BLK = 256


def _gather_body(idx_hbm, pe_hbm, pos_hbm, idx_v, pos_v, gsem):
    cid = lax.axis_index("c")
    sid = lax.axis_index("s")

    @pl.when(jnp.logical_and(cid == 0, sid == 0))
    def _():
        pltpu.sync_copy(idx_hbm, idx_v)
        pltpu.async_copy(pe_hbm.at[idx_v], pos_v, gsem).wait()
        pltpu.sync_copy(pos_v, pos_hbm)


def _sc_gather(idx, pe):
    return pl.kernel(
        _gather_body,
        out_type=jax.ShapeDtypeStruct((BATCH, D_MODEL), jnp.float32),
        mesh=plsc.VectorSubcoreMesh(
            core_axis_name="c", subcore_axis_name="s", num_cores=1),
        scratch_types=[
            pltpu.VMEM((BATCH,), jnp.int32),
            pltpu.VMEM((BATCH, D_MODEL), jnp.float32),
            pltpu.SemaphoreType.DMA,
        ],
    )(idx, pe)


def _add_body(x_ref, pos_ref, o_ref):
    o_ref[...] = x_ref[...] + pos_ref[...][None, :, :]


def kernel(x, pe, idx):
    pos = _sc_gather(idx.astype(jnp.int32), pe)
    return pl.pallas_call(
        _add_body,
        grid=(SEQ_LEN // S_BLK,),
        in_specs=[
            pl.BlockSpec((S_BLK, BATCH, D_MODEL), lambda i: (i, 0, 0)),
            pl.BlockSpec((BATCH, D_MODEL), lambda i: (0, 0)),
        ],
        out_specs=pl.BlockSpec((S_BLK, BATCH, D_MODEL), lambda i: (i, 0, 0)),
        out_shape=jax.ShapeDtypeStruct((SEQ_LEN, BATCH, D_MODEL), jnp.float32),
    )(x, pos)


# speedup vs baseline: 3.5536x; 1.0024x over previous
import jax
import jax.numpy as jnp
from jax import lax
from jax.experimental import pallas as pl
from jax.experimental.pallas import tpu as pltpu
from jax.experimental.pallas import tpu_sc as plsc

D_MODEL = 2048
SEQ_LEN = 4096
BATCH = 4
S_BLK = 256


def _gather_body(idx_hbm, pe_hbm, pos_hbm, idx_v, pos_v, gsem):
    cid = lax.axis_index("c")
    sid = lax.axis_index("s")

    @pl.when(jnp.logical_and(cid == 0, sid == 0))
    def _():
        pltpu.sync_copy(idx_hbm, idx_v)
        pltpu.async_copy(pe_hbm.at[idx_v], pos_v, gsem).wait()
        pltpu.sync_copy(pos_v, pos_hbm)


def _sc_gather(idx, pe):
    return pl.kernel(
        _gather_body,
        out_type=jax.ShapeDtypeStruct((BATCH, D_MODEL), jnp.float32),
        mesh=plsc.VectorSubcoreMesh(
            core_axis_name="c", subcore_axis_name="s", num_cores=1),
        scratch_types=[
            pltpu.VMEM((BATCH,), jnp.int32),
            pltpu.VMEM((BATCH, D_MODEL), jnp.float32),
            pltpu.SemaphoreType.DMA,
        ],
    )(idx, pe)


def _add_body(x_ref, pos_ref, o_ref):
    o_ref[...] = x_ref[...] + pos_ref[...][None, :, :]


def kernel(x, pe, idx):
    pos = _sc_gather(idx.astype(jnp.int32), pe)
    return pl.pallas_call(
        _add_body,
        grid=(SEQ_LEN // S_BLK,),
        in_specs=[
            pl.BlockSpec((S_BLK, BATCH, D_MODEL), lambda i: (i, 0, 0)),
            pl.BlockSpec((BATCH, D_MODEL), lambda i: (0, 0)),
        ],
        out_specs=pl.BlockSpec((S_BLK, BATCH, D_MODEL), lambda i: (i, 0, 0)),
        out_shape=jax.ShapeDtypeStruct((SEQ_LEN, BATCH, D_MODEL), jnp.float32),
    )(x, pos)
